# baseline (speedup 1.0000x reference)
import functools
import math

import jax
import jax.numpy as jnp
from jax.experimental import pallas as pl
from jax.experimental.pallas import tpu as pltpu

_PI = math.pi
_EPS_K = 0.1
_LR_K = 0.1
_SETTLE = 8


def _kuramoto_body(xin_ref, ph0_ref, mask_ref, wd_ref, g5_ref,
                   zr_ref, zi_ref, loss_ref, pscr):
    f = pl.program_id(1)

    x = xin_ref[0, :, 0]
    taps = [x[:, kh:kh + 32, kw:kw + 32]
            for kh in range(3) for kw in range(3)]
    X = jnp.concatenate(taps, axis=0).reshape(27, 32 * 32)
    amp = jnp.dot(wd_ref[...], X, preferred_element_type=jnp.float32)
    amp = jnp.maximum(amp, 0.0).reshape(-1, 32, 32)

    g5 = g5_ref[...]
    bamp = jnp.tanh(amp)

    def update(p):
        cp = jnp.cos(p)
        sp = jnp.sin(p)
        Bc = cp * bamp
        Bs = sp * bamp
        sc_map = Bc.sum(axis=0)
        ss_map = Bs.sum(axis=0)
        Sc = jnp.sum(sc_map)
        Ss = jnp.sum(ss_map)
        scp = jnp.pad(sc_map, 2)
        ssp = jnp.pad(ss_map, 2)
        Cc = jnp.zeros((32, 32), jnp.float32)
        Cs = jnp.zeros((32, 32), jnp.float32)
        for i in range(5):
            for j in range(5):
                w = g5[i, j]
                Cc = Cc + w * scp[i:i + 32, j:j + 32]
                Cs = Cs + w * ssp[i:i + 32, j:j + 32]
        return p + _LR_K * (cp * (Cs - _EPS_K * Ss) - sp * (Cc - _EPS_K * Sc))

    p0 = ph0_ref[0] * (2.0 * _PI) - _PI
    p_init = jnp.where(f == 0, p0, pscr[...])
    p = update(p_init)
    p = jax.lax.cond(
        f == 0,
        lambda q: jax.lax.fori_loop(0, _SETTLE - 1, lambda i, r: update(r), q),
        lambda q: q,
        p)
    pscr[...] = p

    cp = jnp.cos(p)
    sp = jnp.sin(p)
    zr_ref[0, :, 0] = amp * cp
    zi_ref[0, :, 0] = amp * sp

    m = mask_ref[0, 1:, 0].astype(jnp.float32)
    Gm = m.shape[0]
    C = cp.shape[0]
    mflat = m.reshape(Gm, 32 * 32)
    xx = jnp.dot(mflat, cp.reshape(C, 32 * 32).T,
                 preferred_element_type=jnp.float32)
    yy = jnp.dot(mflat, sp.reshape(C, 32 * 32).T,
                 preferred_element_type=jnp.float32)
    gsize = mflat.sum(axis=1)
    gsize = jnp.where(gsize == 0.0, 1.0, gsize)
    go = jnp.sqrt(xx * xx + yy * yy) / gsize[:, None]
    synch = 1.0 - go.mean(axis=-1).sum() / Gm
    mean_ang = jnp.arctan2(yy.mean(axis=-1), xx.mean(axis=-1))
    desynch = jnp.zeros((), jnp.float32)
    for k in range(1, Gm // 2 + 1):
        desynch = desynch + (jnp.cos(k * mean_ang).sum() ** 2 +
                             jnp.sin(k * mean_ang).sum() ** 2) / (2.0 * Gm * k * k)
    lb = 1.0 + 0.5 * Gm * float(
        sum(1.0 / (n * n) for n in range(1, Gm // 2 + 1)))
    loss_ref[...] = ((synch + desynch) / lb).reshape(1, 1, 1, 1)


def _run_kuramoto(xin_pad, phases0, masks, wd2, g5):
    Bn = xin_pad.shape[0]
    Fn = xin_pad.shape[2]
    CK = phases0.shape[1]
    G = masks.shape[1]
    out_shapes = (
        jax.ShapeDtypeStruct((Bn, CK, Fn, 32, 32), jnp.float32),
        jax.ShapeDtypeStruct((Bn, CK, Fn, 32, 32), jnp.float32),
        jax.ShapeDtypeStruct((Bn, Fn, 1, 1), jnp.float32),
    )
    return pl.pallas_call(
        _kuramoto_body,
        grid=(Bn, Fn),
        in_specs=[
            pl.BlockSpec((1, 3, 1, 34, 34), lambda b, f: (b, 0, f, 0, 0)),
            pl.BlockSpec((1, CK, 32, 32), lambda b, f: (b, 0, 0, 0)),
            pl.BlockSpec((1, G, 1, 32, 32), lambda b, f: (b, 0, f, 0, 0)),
            pl.BlockSpec((CK, 27), lambda b, f: (0, 0)),
            pl.BlockSpec((5, 5), lambda b, f: (0, 0)),
        ],
        out_specs=[
            pl.BlockSpec((1, CK, 1, 32, 32), lambda b, f: (b, 0, f, 0, 0)),
            pl.BlockSpec((1, CK, 1, 32, 32), lambda b, f: (b, 0, f, 0, 0)),
            pl.BlockSpec((1, 1, 1, 1), lambda b, f: (b, f, 0, 0)),
        ],
        out_shape=out_shapes,
        scratch_shapes=[pltpu.VMEM((CK, 32, 32), jnp.float32)],
        compiler_params=pltpu.CompilerParams(
            dimension_semantics=("parallel", "arbitrary")),
    )(xin_pad, phases0, masks, wd2, g5)


def _split_eo(x):
    xp = jnp.pad(x, ((0, 0), (0, 0), (1, 1), (1, 1), (1, 1)))
    Bn, C, Dp, Hp, Wp = xp.shape
    x6 = xp.reshape(Bn, C, Dp, Hp // 2, 2, Wp // 2, 2)
    return x6.transpose(0, 1, 2, 4, 6, 3, 5)


def _cconv_body(so, act, x0r_ref, x1r_ref, x2r_ref, x0i_ref, x1i_ref,
                x2i_ref, w2_ref, br_ref, bi_ref, scale_ref, shift_ref,
                zr_ref, zi_ref, psum_ref, acc_scr):
    C = x0r_ref.shape[1]
    O = zr_ref.shape[1]
    so2 = so * so
    d = pl.program_id(1)
    nd = pl.num_programs(1)

    vr = [x0r_ref[0, :, 0], x1r_ref[0, :, 0], x2r_ref[0, :, 0]]
    vi = [x0i_ref[0, :, 0], x1i_ref[0, :, 0], x2i_ref[0, :, 0]]
    if act:
        scale = scale_ref[...].reshape(C, 1, 1, 1, 1)
        shift = shift_ref[...].reshape(C, 1, 1, 1, 1)
        for k in range(3):
            xr, xi = vr[k], vi[k]
            mag = jnp.sqrt(xr * xr + xi * xi + 1e-12)
            mact = jnp.maximum(mag * scale + shift, 0.0)
            ang = jnp.arctan2(xi, xr)
            vr[k] = mact * jnp.cos(ang)
            vi[k] = mact * jnp.sin(ang)

    tr = []
    ti = []
    for kd in range(3):
        for kh in range(3):
            for kw in range(3):
                ph, oh = kh % 2, kh // 2
                pw, ow = kw % 2, kw // 2
                tr.append(vr[kd][:, ph, pw, oh:oh + so, ow:ow + so]
                          .reshape(C, so2))
                ti.append(vi[kd][:, ph, pw, oh:oh + so, ow:ow + so]
                          .reshape(C, so2))
    Xri = jnp.concatenate(
        [jnp.concatenate(tr, axis=0), jnp.concatenate(ti, axis=0)],
        axis=1)
    out = jnp.dot(w2_ref[...], Xri, preferred_element_type=jnp.float32)
    outr = out[:, :so2] + br_ref[...].reshape(O, 1)
    outi = out[:, so2:] + bi_ref[...].reshape(O, 1)
    zr_ref[0, :, 0] = outr.reshape(O, so, so)
    zi_ref[0, :, 0] = outi.reshape(O, so, so)

    mag = jnp.sqrt(outr * outr + outi * outi + 1e-12)

    @pl.when(d == 0)
    def _():
        acc_scr[...] = jnp.zeros_like(acc_scr)

    acc_scr[0, :] += mag.sum(axis=1)
    acc_scr[1, :] += (mag * mag).sum(axis=1)

    @pl.when(d == nd - 1)
    def _():
        psum_ref[0] = acc_scr[...]


def _run_cconv(xr, xi, w, br, bi, scale, shift, so):
    act = scale is not None
    Bn, C, D = xr.shape[0], xr.shape[1], xr.shape[2]
    O = w.shape[0]
    xer = _split_eo(xr)
    xei = _split_eo(xi)
    _, _, Dp, _, _, hh, ww = xer.shape
    w2 = w.transpose(0, 2, 3, 4, 1).reshape(O, 27 * C)
    if not act:
        scale = jnp.ones((C, 1), jnp.float32)
        shift = jnp.zeros((C, 1), jnp.float32)
    out_shapes = (
        jax.ShapeDtypeStruct((Bn, O, D, so, so), jnp.float32),
        jax.ShapeDtypeStruct((Bn, O, D, so, so), jnp.float32),
        jax.ShapeDtypeStruct((Bn, 2, O), jnp.float32),
    )
    body = functools.partial(_cconv_body, so, act)

    def xspec(k):
        return pl.BlockSpec((1, C, 1, 2, 2, hh, ww),
                            lambda b, d, k=k: (b, 0, d + k, 0, 0, 0, 0))

    return pl.pallas_call(
        body,
        grid=(Bn, D),
        in_specs=[
            xspec(0), xspec(1), xspec(2),
            xspec(0), xspec(1), xspec(2),
            pl.BlockSpec((O, 27 * C), lambda b, d: (0, 0)),
            pl.BlockSpec((O, 1), lambda b, d: (0, 0)),
            pl.BlockSpec((O, 1), lambda b, d: (0, 0)),
            pl.BlockSpec((C, 1), lambda b, d: (0, 0)),
            pl.BlockSpec((C, 1), lambda b, d: (0, 0)),
        ],
        out_specs=[
            pl.BlockSpec((1, O, 1, so, so), lambda b, d: (b, 0, d, 0, 0)),
            pl.BlockSpec((1, O, 1, so, so), lambda b, d: (b, 0, d, 0, 0)),
            pl.BlockSpec((1, 2, O), lambda b, d: (b, 0, 0)),
        ],
        out_shape=out_shapes,
        scratch_shapes=[pltpu.VMEM((2, O), jnp.float32)],
        compiler_params=pltpu.CompilerParams(
            dimension_semantics=("parallel", "arbitrary")),
    )(xer, xer, xer, xei, xei, xei, w2, br.reshape(O, 1), bi.reshape(O, 1),
      scale, shift)


def _bn_finalize(psum, g, be, n):
    mean = psum[:, 0].sum(axis=0) / n
    ex2 = psum[:, 1].sum(axis=0) / n
    var = ex2 - mean * mean
    scale = g * jax.lax.rsqrt(var + 1e-3)
    shift = be - mean * scale
    return scale.reshape(-1, 1), shift.reshape(-1, 1)


def _head_body(xr_ref, xi_ref, wo_ref, scale_ref, shift_ref, bo_ref,
               out_ref):
    C = xr_ref.shape[1]
    xr = xr_ref[0]
    xi = xi_ref[0]
    scale = scale_ref[...].reshape(C, 1)
    shift = shift_ref[...].reshape(C, 1)
    mag = jnp.sqrt(xr * xr + xi * xi + 1e-12)
    mact = jnp.maximum(mag * scale + shift, 0.0)
    ang = jnp.arctan2(xi, xr)
    ar = mact * jnp.cos(ang)
    ai = mact * jnp.sin(ang)
    w = wo_ref[...]
    orv = (ar * w).sum() + bo_ref[0, 0]
    oiv = (ai * w).sum() + bo_ref[0, 1]
    out_ref[...] = jnp.sqrt(orv * orv + oiv * oiv + 1e-12).reshape(1, 1, 1)


def _run_head(zr, zi, wo, scale, shift, bo):
    Bn, C, N = zr.shape
    return pl.pallas_call(
        _head_body,
        grid=(Bn,),
        in_specs=[
            pl.BlockSpec((1, C, N), lambda b: (b, 0, 0)),
            pl.BlockSpec((1, C, N), lambda b: (b, 0, 0)),
            pl.BlockSpec((C, N), lambda b: (0, 0)),
            pl.BlockSpec((C, 1), lambda b: (0, 0)),
            pl.BlockSpec((C, 1), lambda b: (0, 0)),
            pl.BlockSpec((1, 2), lambda b: (0, 0)),
        ],
        out_specs=pl.BlockSpec((1, 1, 1), lambda b: (b, 0, 0)),
        out_shape=jax.ShapeDtypeStruct((Bn, 1, 1), jnp.float32),
        compiler_params=pltpu.CompilerParams(
            dimension_semantics=("parallel",)),
    )(zr, zi, wo, scale, shift, bo)


def kernel(input, masks, phases0, w_down, kernel_kuramoto, w_pre, b_pre_r,
           b_pre_i, g1, be1, w_c1, b_c1_r, b_c1_i, g2, be2, w_c2, b_c2_r,
           b_c2_i, g3, be3, w_out, b_out_r, b_out_i):
    Bn, _, Fn, Hn, Wn = input.shape
    CK = phases0.shape[1]

    xin_pad = jnp.pad(input, ((0, 0), (0, 0), (0, 0), (1, 1), (1, 1)))
    wd2 = w_down.transpose(0, 2, 3, 1).reshape(CK, 27)
    g5 = kernel_kuramoto[0, 0]

    zr0, zi0, losses = _run_kuramoto(xin_pad, phases0, masks, wd2, g5)
    loss_synch = losses[:, :, 0, 0].mean(axis=0).sum() / Fn

    zr1, zi1, ps1 = _run_cconv(zr0, zi0, w_pre, b_pre_r, b_pre_i,
                               None, None, 16)
    sc1, sh1 = _bn_finalize(ps1, g1, be1, Bn * Fn * 16 * 16)
    zr2, zi2, ps2 = _run_cconv(zr1, zi1, w_c1, b_c1_r, b_c1_i, sc1, sh1, 8)
    sc2, sh2 = _bn_finalize(ps2, g2, be2, Bn * Fn * 8 * 8)
    zr3, zi3, ps3 = _run_cconv(zr2, zi2, w_c2, b_c2_r, b_c2_i, sc2, sh2, 4)
    sc3, sh3 = _bn_finalize(ps3, g3, be3, Bn * Fn * 4 * 4)

    Co = zr3.shape[1]
    wo = w_out[0].reshape(Co, Fn * 16)
    bo = jnp.stack([b_out_r[0], b_out_i[0]]).reshape(1, 2)
    x = _run_head(zr3.reshape(Bn, Co, Fn * 16), zi3.reshape(Bn, Co, Fn * 16),
                  wo, sc3, sh3, bo)[:, :, 0]
    return x, loss_synch
